# Initial kernel scaffold; baseline (speedup 1.0000x reference)
import functools

import jax
import jax.numpy as jnp
from jax import lax
from jax.experimental import pallas as pl
from jax.experimental.pallas import tpu as pltpu
from jax.experimental.pallas import tpu_sc as plsc

N = 10000
E = 160000
D = 256
R = 7
L = 3
H = 128
NTILES = 16
ET = E // NTILES
B = 125
NB = ET // B
ROWS_PT = N // NTILES


def _y_body(h_ref, w_ref, y_ref):
    acc = jnp.dot(h_ref[...], w_ref[0], preferred_element_type=jnp.float32)
    y_ref[0] = acc[:, :H]
    y_ref[1] = acc[:, H:]


def _tc_relation_table(h, w_rel):
    bn = 2000
    nb = N // bn
    return pl.pallas_call(
        _y_body,
        grid=(nb, R),
        in_specs=[
            pl.BlockSpec((bn, D), lambda i, r: (i, 0)),
            pl.BlockSpec((1, D, D), lambda i, r: (r, 0, 0)),
        ],
        out_specs=pl.BlockSpec((2, bn, H), lambda i, r: (0, r * nb + i, 0)),
        out_shape=jax.ShapeDtypeStruct((2, R * N, H), jnp.float32),
    )(h, w_rel)


def _z_body(h_ref, w_ref, b_ref, z_ref):
    acc = jnp.dot(h_ref[...], w_ref[...], preferred_element_type=jnp.float32)
    acc = acc + b_ref[...]
    z_ref[0] = acc[:, :H]
    z_ref[1] = acc[:, H:]


def _tc_self_term(h, w_self, b):
    bn = 2000
    nb = N // bn
    return pl.pallas_call(
        _z_body,
        grid=(nb,),
        in_specs=[
            pl.BlockSpec((bn, D), lambda i: (i, 0)),
            pl.BlockSpec((D, D), lambda i: (0, 0)),
            pl.BlockSpec((1, D), lambda i: (0, 0)),
        ],
        out_specs=pl.BlockSpec((2, bn, H), lambda i: (0, i, 0)),
        out_shape=jax.ShapeDtypeStruct((2, N, H), jnp.float32),
    )(h, w_self, b.reshape(1, D))


def _combine_body(a_ref, o_ref):
    o_ref[...] = jnp.maximum(
        jnp.concatenate([a_ref[0], a_ref[1]], axis=-1), 0.0)


def _tc_relu_concat(acc2):
    bn = 2000
    nb = N // bn
    return pl.pallas_call(
        _combine_body,
        grid=(nb,),
        in_specs=[pl.BlockSpec((2, bn, H), lambda i: (0, i, 0))],
        out_specs=pl.BlockSpec((bn, D), lambda i: (i, 0)),
        out_shape=jax.ShapeDtypeStruct((N, D), jnp.float32),
    )(acc2)


def _sc_scatter_body(y_hbm, z_hbm, gidx_hbm, dst_hbm, out_hbm,
                     gidx_v, dst_v, buf, acc):
    c = lax.axis_index("c")
    s = lax.axis_index("s")
    pltpu.sync_copy(z_hbm.at[c, pl.ds(s * ROWS_PT, ROWS_PT)],
                    acc.at[pl.ds(s * ROWS_PT, ROWS_PT)])
    pltpu.sync_copy(gidx_hbm.at[c, s], gidx_v)
    pltpu.sync_copy(dst_hbm.at[s], dst_v)
    plsc.subcore_barrier()

    def body(j, carry):
        pltpu.sync_copy(y_hbm.at[gidx_v.at[j]], buf)
        pltpu.sync_copy(buf, acc.at[dst_v.at[j]], add=True)
        return carry

    lax.fori_loop(0, NB, body, 0)
    plsc.subcore_barrier()
    pltpu.sync_copy(acc.at[pl.ds(s * ROWS_PT, ROWS_PT)],
                    out_hbm.at[c, pl.ds(s * ROWS_PT, ROWS_PT)])


@functools.partial(
    pl.kernel,
    out_type=jax.ShapeDtypeStruct((2, N, H), jnp.float32),
    mesh=plsc.VectorSubcoreMesh(core_axis_name="c", subcore_axis_name="s"),
    scratch_types=[
        pltpu.VMEM((NB, B), jnp.int32),
        pltpu.VMEM((NB, B), jnp.int32),
        pltpu.VMEM((B, H), jnp.float32),
        pltpu.VMEM_SHARED((N, H), jnp.float32),
    ],
)
def _sc_scatter(y_hbm, z_hbm, gidx_hbm, dst_hbm, out_hbm,
                gidx_v, dst_v, buf, acc):
    _sc_scatter_body(y_hbm, z_hbm, gidx_hbm, dst_hbm, out_hbm,
                     gidx_v, dst_v, buf, acc)


def kernel(x, edge_index, edge_type, node_position, W_rel, W_self, b):
    src = edge_index[0]
    dst = edge_index[1]
    tidx = edge_type * N + src
    gidx = jnp.stack([tidx, tidx + R * N]).reshape(2, NTILES, NB, B)
    dst_r = dst.reshape(NTILES, NB, B)

    h = x
    outs = []
    for l in range(L):
        y = _tc_relation_table(h, W_rel[l]).reshape(2 * R * N, H)
        z = _tc_self_term(h, W_self[l], b[l])
        acc2 = _sc_scatter(y, z, gidx, dst_r)
        h = _tc_relu_concat(acc2)
        outs.append(h)
    node_feature = jnp.concatenate(outs, axis=-1)
    return node_feature, node_position


# speedup vs baseline: 4.9156x; 4.9156x over previous
import functools

import jax
import jax.numpy as jnp
from jax import lax
from jax.experimental import pallas as pl
from jax.experimental.pallas import tpu as pltpu
from jax.experimental.pallas import tpu_sc as plsc

N = 10000
E = 160000
D = 256
R = 7
L = 3
H = 128
NTILES = 16
ET = E // NTILES
B = 125
NB = ET // B
CHUNK = 640
LAST = N - CHUNK * (NTILES - 1)


def _y_body(h_ref, w_ref, y_ref):
    acc = jnp.dot(h_ref[...], w_ref[0], preferred_element_type=jnp.float32)
    y_ref[0] = acc[:, :H]
    y_ref[1] = acc[:, H:]


def _tc_relation_table(h, w_rel):
    bn = 2000
    nb = N // bn
    return pl.pallas_call(
        _y_body,
        grid=(nb, R),
        in_specs=[
            pl.BlockSpec((bn, D), lambda i, r: (i, 0)),
            pl.BlockSpec((1, D, D), lambda i, r: (r, 0, 0)),
        ],
        out_specs=pl.BlockSpec((2, bn, H), lambda i, r: (0, r * nb + i, 0)),
        out_shape=jax.ShapeDtypeStruct((2, R * N, H), jnp.float32),
    )(h, w_rel)


def _z_body(h_ref, w_ref, b_ref, z_ref):
    acc = jnp.dot(h_ref[...], w_ref[...], preferred_element_type=jnp.float32)
    acc = acc + b_ref[...]
    z_ref[0] = acc[:, :H]
    z_ref[1] = acc[:, H:]


def _tc_self_term(h, w_self, b):
    bn = 2000
    nb = N // bn
    return pl.pallas_call(
        _z_body,
        grid=(nb,),
        in_specs=[
            pl.BlockSpec((bn, D), lambda i: (i, 0)),
            pl.BlockSpec((D, D), lambda i: (0, 0)),
            pl.BlockSpec((1, D), lambda i: (0, 0)),
        ],
        out_specs=pl.BlockSpec((2, bn, H), lambda i: (0, i, 0)),
        out_shape=jax.ShapeDtypeStruct((2, N, H), jnp.float32),
    )(h, w_self, b.reshape(1, D))


def _combine_body(a_ref, o_ref):
    o_ref[...] = jnp.maximum(
        jnp.concatenate([a_ref[0], a_ref[1]], axis=-1), 0.0)


def _tc_relu_concat(acc2):
    bn = 2000
    nb = N // bn
    return pl.pallas_call(
        _combine_body,
        grid=(nb,),
        in_specs=[pl.BlockSpec((2, bn, H), lambda i: (0, i, 0))],
        out_specs=pl.BlockSpec((bn, D), lambda i: (i, 0)),
        out_shape=jax.ShapeDtypeStruct((N, D), jnp.float32),
    )(acc2)


def _sc_scatter_body(y_hbm, z_hbm, gidx_hbm, dst_hbm, out_hbm,
                     gidx_v, dst_v, buf, acc):
    c = lax.axis_index("c")
    s = lax.axis_index("s")
    @pl.when(s < NTILES - 1)
    def _():
        pltpu.sync_copy(z_hbm.at[c, pl.ds(s * CHUNK, CHUNK)],
                        acc.at[pl.ds(s * CHUNK, CHUNK)])

    @pl.when(s == NTILES - 1)
    def _():
        pltpu.sync_copy(z_hbm.at[c, pl.ds((NTILES - 1) * CHUNK, LAST)],
                        acc.at[pl.ds((NTILES - 1) * CHUNK, LAST)])
    pltpu.sync_copy(gidx_hbm.at[c, s], gidx_v)
    pltpu.sync_copy(dst_hbm.at[s], dst_v)
    plsc.subcore_barrier()

    def body(j, carry):
        pltpu.sync_copy(y_hbm.at[gidx_v.at[j]], buf)
        pltpu.sync_copy(buf, acc.at[dst_v.at[j]], add=True)
        return carry

    lax.fori_loop(0, NB, body, 0)
    plsc.subcore_barrier()

    @pl.when(s < NTILES - 1)
    def _():
        pltpu.sync_copy(acc.at[pl.ds(s * CHUNK, CHUNK)],
                        out_hbm.at[c, pl.ds(s * CHUNK, CHUNK)])

    @pl.when(s == NTILES - 1)
    def _():
        pltpu.sync_copy(acc.at[pl.ds((NTILES - 1) * CHUNK, LAST)],
                        out_hbm.at[c, pl.ds((NTILES - 1) * CHUNK, LAST)])


@functools.cache
def _make_sc_scatter():
    return pl.kernel(
        _sc_scatter_body,
        out_type=jax.ShapeDtypeStruct((2, N, H), jnp.float32),
        mesh=plsc.VectorSubcoreMesh(core_axis_name="c", subcore_axis_name="s",
                                    num_cores=2, num_subcores=NTILES),
        scratch_types=[
            pltpu.VMEM((NB, B), jnp.int32),
            pltpu.VMEM((NB, B), jnp.int32),
            pltpu.VMEM((B, H), jnp.float32),
            pltpu.VMEM_SHARED((N, H), jnp.float32),
        ],
    )


def kernel(x, edge_index, edge_type, node_position, W_rel, W_self, b):
    src = edge_index[0]
    dst = edge_index[1]
    tidx = edge_type * N + src
    gidx = jnp.stack([tidx, tidx + R * N]).reshape(2, NTILES, NB, B)
    dst_r = dst.reshape(NTILES, NB, B)

    h = x
    outs = []
    for l in range(L):
        y = _tc_relation_table(h, W_rel[l]).reshape(2 * R * N, H)
        z = _tc_self_term(h, W_self[l], b[l])
        acc2 = _make_sc_scatter()(y, z, gidx, dst_r)
        h = _tc_relu_concat(acc2)
        outs.append(h)
    node_feature = jnp.concatenate(outs, axis=-1)
    return node_feature, node_position


# speedup vs baseline: 6.2455x; 1.2706x over previous
import functools

import jax
import jax.numpy as jnp
from jax import lax
from jax.experimental import pallas as pl
from jax.experimental.pallas import tpu as pltpu
from jax.experimental.pallas import tpu_sc as plsc

N = 10000
E = 160000
D = 256
R = 7
L = 3
H = 128
NTILES = 16
ET = E // NTILES
B = 125
NB = ET // B
SB = 10
NSB = NB // SB
CHUNK = 640
LAST = N - CHUNK * (NTILES - 1)


def _y_body(h_ref, w_ref, y_ref):
    acc = jnp.dot(h_ref[...], w_ref[0], preferred_element_type=jnp.float32)
    y_ref[0] = acc[:, :H]
    y_ref[1] = acc[:, H:]


def _tc_relation_table(h, w_rel):
    bn = 2000
    nb = N // bn
    return pl.pallas_call(
        _y_body,
        grid=(nb, R),
        in_specs=[
            pl.BlockSpec((bn, D), lambda i, r: (i, 0)),
            pl.BlockSpec((1, D, D), lambda i, r: (r, 0, 0)),
        ],
        out_specs=pl.BlockSpec((2, bn, H), lambda i, r: (0, r * nb + i, 0)),
        out_shape=jax.ShapeDtypeStruct((2, R * N, H), jnp.float32),
    )(h, w_rel)


def _z_body(h_ref, w_ref, b_ref, z_ref):
    acc = jnp.dot(h_ref[...], w_ref[...], preferred_element_type=jnp.float32)
    acc = acc + b_ref[...]
    z_ref[0] = acc[:, :H]
    z_ref[1] = acc[:, H:]


def _tc_self_term(h, w_self, b):
    bn = 2000
    nb = N // bn
    return pl.pallas_call(
        _z_body,
        grid=(nb,),
        in_specs=[
            pl.BlockSpec((bn, D), lambda i: (i, 0)),
            pl.BlockSpec((D, D), lambda i: (0, 0)),
            pl.BlockSpec((1, D), lambda i: (0, 0)),
        ],
        out_specs=pl.BlockSpec((2, bn, H), lambda i: (0, i, 0)),
        out_shape=jax.ShapeDtypeStruct((2, N, H), jnp.float32),
    )(h, w_self, b.reshape(1, D))


def _combine_body(a_ref, o_ref):
    o_ref[...] = jnp.maximum(
        jnp.concatenate([a_ref[0], a_ref[1]], axis=-1), 0.0)


def _tc_relu_concat(acc2):
    bn = 2000
    nb = N // bn
    return pl.pallas_call(
        _combine_body,
        grid=(nb,),
        in_specs=[pl.BlockSpec((2, bn, H), lambda i: (0, i, 0))],
        out_specs=pl.BlockSpec((bn, D), lambda i: (i, 0)),
        out_shape=jax.ShapeDtypeStruct((N, D), jnp.float32),
    )(acc2)


NBUF = 2


def _sc_scatter_body(y_hbm, z_hbm, idx_hbm, out_hbm,
                     ibufs, isems, bufs, sems, acc):
    c = lax.axis_index("c")
    s = lax.axis_index("s")
    @pl.when(s < NTILES - 1)
    def _():
        pltpu.sync_copy(z_hbm.at[c, pl.ds(s * CHUNK, CHUNK)],
                        acc.at[pl.ds(s * CHUNK, CHUNK)])

    @pl.when(s == NTILES - 1)
    def _():
        pltpu.sync_copy(z_hbm.at[c, pl.ds((NTILES - 1) * CHUNK, LAST)],
                        acc.at[pl.ds((NTILES - 1) * CHUNK, LAST)])
    pltpu.async_copy(idx_hbm.at[c, s, 0], ibufs[0], isems[0])
    plsc.subcore_barrier()

    for k in range(NSB):
        ib = ibufs[k % 2]
        pltpu.make_async_copy(idx_hbm.at[c, s, k], ib, isems[k % 2]).wait()
        if k + 1 < NSB:
            pltpu.async_copy(idx_hbm.at[c, s, k + 1],
                             ibufs[(k + 1) % 2], isems[(k + 1) % 2])
        for p in range(NBUF):
            pltpu.async_copy(y_hbm.at[ib.at[p, 0]], bufs[p], sems[p])

        def inner(b2, carry, ib=ib):
            for p in range(NBUF):
                bl = b2 * NBUF + p
                pltpu.make_async_copy(y_hbm.at[ib.at[bl, 0]],
                                      bufs[p], sems[p]).wait()
                pltpu.sync_copy(bufs[p], acc.at[ib.at[bl, 1]], add=True)

                @pl.when(bl + NBUF < SB)
                def _():
                    pltpu.async_copy(y_hbm.at[ib.at[bl + NBUF, 0]],
                                     bufs[p], sems[p])
            return carry

        lax.fori_loop(0, SB // NBUF, inner, 0)
    plsc.subcore_barrier()

    @pl.when(s < NTILES - 1)
    def _():
        pltpu.sync_copy(acc.at[pl.ds(s * CHUNK, CHUNK)],
                        out_hbm.at[c, pl.ds(s * CHUNK, CHUNK)])

    @pl.when(s == NTILES - 1)
    def _():
        pltpu.sync_copy(acc.at[pl.ds((NTILES - 1) * CHUNK, LAST)],
                        out_hbm.at[c, pl.ds((NTILES - 1) * CHUNK, LAST)])


@functools.cache
def _make_sc_scatter():
    return pl.kernel(
        _sc_scatter_body,
        out_type=jax.ShapeDtypeStruct((2, N, H), jnp.float32),
        mesh=plsc.VectorSubcoreMesh(core_axis_name="c", subcore_axis_name="s",
                                    num_cores=2, num_subcores=NTILES),
        scratch_types=[
            tuple(pltpu.VMEM((SB, 2, B), jnp.int32) for _ in range(2)),
            tuple(pltpu.SemaphoreType.DMA for _ in range(2)),
            tuple(pltpu.VMEM((B, H), jnp.float32) for _ in range(NBUF)),
            tuple(pltpu.SemaphoreType.DMA for _ in range(NBUF)),
            pltpu.VMEM_SHARED((N, H), jnp.float32),
        ],
    )


def kernel(x, edge_index, edge_type, node_position, W_rel, W_self, b):
    src = edge_index[0]
    dst = edge_index[1]
    tidx = edge_type * N + src
    ge = jnp.stack([tidx, tidx + R * N]).reshape(2, NTILES, NSB, SB, B)
    de = jnp.broadcast_to(dst, (2, E)).reshape(2, NTILES, NSB, SB, B)
    idx = jnp.stack([ge, de], axis=-2)

    h = x
    outs = []
    for l in range(L):
        y = _tc_relation_table(h, W_rel[l]).reshape(2 * R * N, H)
        z = _tc_self_term(h, W_self[l], b[l])
        acc2 = _make_sc_scatter()(y, z, idx)
        h = _tc_relu_concat(acc2)
        outs.append(h)
    node_feature = jnp.concatenate(outs, axis=-1)
    return node_feature, node_position


# speedup vs baseline: 6.7076x; 1.0740x over previous
import functools

import jax
import jax.numpy as jnp
from jax import lax
from jax.experimental import pallas as pl
from jax.experimental.pallas import tpu as pltpu
from jax.experimental.pallas import tpu_sc as plsc

N = 10000
E = 160000
D = 256
R = 7
L = 3
H = 128
NTILES = 16
ET = E // NTILES
B = 125
NB = ET // B
SB = 10
NSB = NB // SB
CHUNK = 640
LAST = N - CHUNK * (NTILES - 1)


NROW = (R + 1) * N


def _yz_body(h_ref, w_ref, b_ref, y_ref):
    acc = jnp.dot(h_ref[...], w_ref[0], preferred_element_type=jnp.float32)
    sel = jnp.where(pl.program_id(1) == R, 1.0, 0.0)
    acc = acc + sel * b_ref[...]
    y_ref[0] = acc[:, :H]
    y_ref[1] = acc[:, H:]


def _tc_tables(h, w_cat, b):
    bn = 2000
    nb = N // bn
    return pl.pallas_call(
        _yz_body,
        grid=(nb, R + 1),
        in_specs=[
            pl.BlockSpec((bn, D), lambda i, r: (i, 0)),
            pl.BlockSpec((1, D, D), lambda i, r: (r, 0, 0)),
            pl.BlockSpec((1, D), lambda i, r: (0, 0)),
        ],
        out_specs=pl.BlockSpec((2, bn, H), lambda i, r: (0, r * nb + i, 0)),
        out_shape=jax.ShapeDtypeStruct((2, NROW, H), jnp.float32),
    )(h, w_cat, b.reshape(1, D))


def _combine_body(a_ref, o_ref):
    o_ref[...] = jnp.maximum(
        jnp.concatenate([a_ref[0], a_ref[1]], axis=-1), 0.0)


def _tc_relu_concat(acc2):
    bn = 2000
    nb = N // bn
    return pl.pallas_call(
        _combine_body,
        grid=(nb,),
        in_specs=[pl.BlockSpec((2, bn, H), lambda i: (0, i, 0))],
        out_specs=pl.BlockSpec((bn, D), lambda i: (i, 0)),
        out_shape=jax.ShapeDtypeStruct((N, D), jnp.float32),
    )(acc2)


NBUF = 2


def _sc_scatter_body(y_hbm, idx_hbm, out_hbm,
                     ibufs, isems, bufs, sems, acc):
    c = lax.axis_index("c")
    s = lax.axis_index("s")
    pltpu.async_copy(idx_hbm.at[c, s, 0], ibufs[0], isems[0])
    zbase = c * NROW + R * N

    @pl.when(s < NTILES - 1)
    def _():
        pltpu.sync_copy(y_hbm.at[pl.ds(zbase + s * CHUNK, CHUNK)],
                        acc.at[pl.ds(s * CHUNK, CHUNK)])

    @pl.when(s == NTILES - 1)
    def _():
        pltpu.sync_copy(
            y_hbm.at[pl.ds(zbase + (NTILES - 1) * CHUNK, LAST)],
            acc.at[pl.ds((NTILES - 1) * CHUNK, LAST)])

    pltpu.make_async_copy(idx_hbm.at[c, s, 0], ibufs[0], isems[0]).wait()
    pltpu.async_copy(idx_hbm.at[c, s, 1], ibufs[1], isems[1])
    plsc.subcore_barrier()

    for p in range(NBUF):
        pltpu.async_copy(y_hbm.at[ibufs[0].at[p, 0]], bufs[p], sems[p])

    for k in range(NSB):
        ib = ibufs[k % 2]
        ibn = ibufs[(k + 1) % 2]

        def inner(b2, carry, ib=ib):
            for p in range(NBUF):
                bl = b2 * NBUF + p
                pltpu.make_async_copy(y_hbm.at[ib.at[bl, 0]],
                                      bufs[p], sems[p]).wait()
                pltpu.sync_copy(bufs[p], acc.at[ib.at[bl, 1]], add=True)
                pltpu.async_copy(y_hbm.at[ib.at[bl + NBUF, 0]],
                                 bufs[p], sems[p])
            return carry

        lax.fori_loop(0, (SB - NBUF) // NBUF, inner, 0)
        if k + 1 < NSB:
            pltpu.make_async_copy(idx_hbm.at[c, s, k + 1], ibn,
                                  isems[(k + 1) % 2]).wait()
        for p in range(NBUF):
            bl = SB - NBUF + p
            pltpu.make_async_copy(y_hbm.at[ib.at[bl, 0]],
                                  bufs[p], sems[p]).wait()
            pltpu.sync_copy(bufs[p], acc.at[ib.at[bl, 1]], add=True)
            if k + 1 < NSB:
                pltpu.async_copy(y_hbm.at[ibn.at[p, 0]], bufs[p], sems[p])
        if k + 2 < NSB:
            pltpu.async_copy(idx_hbm.at[c, s, k + 2], ib, isems[k % 2])
    plsc.subcore_barrier()

    @pl.when(s < NTILES - 1)
    def _():
        pltpu.sync_copy(acc.at[pl.ds(s * CHUNK, CHUNK)],
                        out_hbm.at[c, pl.ds(s * CHUNK, CHUNK)])

    @pl.when(s == NTILES - 1)
    def _():
        pltpu.sync_copy(acc.at[pl.ds((NTILES - 1) * CHUNK, LAST)],
                        out_hbm.at[c, pl.ds((NTILES - 1) * CHUNK, LAST)])


@functools.cache
def _make_sc_scatter():
    return pl.kernel(
        _sc_scatter_body,
        out_type=jax.ShapeDtypeStruct((2, N, H), jnp.float32),
        mesh=plsc.VectorSubcoreMesh(core_axis_name="c", subcore_axis_name="s",
                                    num_cores=2, num_subcores=NTILES),
        scratch_types=[
            tuple(pltpu.VMEM((SB, 2, B), jnp.int32) for _ in range(2)),
            tuple(pltpu.SemaphoreType.DMA for _ in range(2)),
            tuple(pltpu.VMEM((B, H), jnp.float32) for _ in range(NBUF)),
            tuple(pltpu.SemaphoreType.DMA for _ in range(NBUF)),
            pltpu.VMEM_SHARED((N, H), jnp.float32),
        ],
    )


def kernel(x, edge_index, edge_type, node_position, W_rel, W_self, b):
    src = edge_index[0]
    dst = edge_index[1]
    tidx = edge_type * N + src
    ge = jnp.stack([tidx, tidx + NROW]).reshape(2, NTILES, NSB, SB, B)
    de = jnp.broadcast_to(dst, (2, E)).reshape(2, NTILES, NSB, SB, B)
    idx = jnp.stack([ge, de], axis=-2)
    w_cat = jnp.concatenate([W_rel, W_self[:, None]], axis=1)

    h = x
    outs = []
    for l in range(L):
        yz = _tc_tables(h, w_cat[l], b[l])
        acc2 = _make_sc_scatter()(yz.reshape(2 * NROW, H), idx)
        h = _tc_relu_concat(acc2)
        outs.append(h)
    node_feature = jnp.concatenate(outs, axis=-1)
    return node_feature, node_position


# speedup vs baseline: 6.7083x; 1.0001x over previous
import functools

import jax
import jax.numpy as jnp
from jax import lax
from jax.experimental import pallas as pl
from jax.experimental.pallas import tpu as pltpu
from jax.experimental.pallas import tpu_sc as plsc

N = 10000
E = 160000
D = 256
R = 7
L = 3
H = 128
NTILES = 16
ET = E // NTILES
B = 125
NB = ET // B
SB = 10
NSB = NB // SB
CHUNK = 640
LAST = N - CHUNK * (NTILES - 1)


NROW = (R + 1) * N


def _yz_body(h_ref, w_ref, b_ref, y_ref):
    acc = jnp.dot(h_ref[...].astype(jnp.bfloat16),
                  w_ref[0].astype(jnp.bfloat16),
                  preferred_element_type=jnp.float32)
    sel = jnp.where(pl.program_id(1) == R, 1.0, 0.0)
    acc = acc + sel * b_ref[...]
    y_ref[0] = acc[:, :H]
    y_ref[1] = acc[:, H:]


def _tc_tables(h, w_cat, b):
    bn = 2000
    nb = N // bn
    return pl.pallas_call(
        _yz_body,
        grid=(nb, R + 1),
        in_specs=[
            pl.BlockSpec((bn, D), lambda i, r: (i, 0)),
            pl.BlockSpec((1, D, D), lambda i, r: (r, 0, 0)),
            pl.BlockSpec((1, D), lambda i, r: (0, 0)),
        ],
        out_specs=pl.BlockSpec((2, bn, H), lambda i, r: (0, r * nb + i, 0)),
        out_shape=jax.ShapeDtypeStruct((2, NROW, H), jnp.float32),
    )(h, w_cat, b.reshape(1, D))


def _combine_body(a_ref, o_ref):
    o_ref[...] = jnp.maximum(
        jnp.concatenate([a_ref[0], a_ref[1]], axis=-1), 0.0)


def _tc_relu_concat(acc2):
    bn = 2000
    nb = N // bn
    return pl.pallas_call(
        _combine_body,
        grid=(nb,),
        in_specs=[pl.BlockSpec((2, bn, H), lambda i: (0, i, 0))],
        out_specs=pl.BlockSpec((bn, D), lambda i: (i, 0)),
        out_shape=jax.ShapeDtypeStruct((N, D), jnp.float32),
    )(acc2)


NBUF = 2


def _sc_scatter_body(y_hbm, idx_hbm, out_hbm,
                     ibufs, isems, bufs, sems, acc):
    c = lax.axis_index("c")
    s = lax.axis_index("s")
    pltpu.async_copy(idx_hbm.at[c, s, 0], ibufs[0], isems[0])
    zbase = c * NROW + R * N

    @pl.when(s < NTILES - 1)
    def _():
        pltpu.sync_copy(y_hbm.at[pl.ds(zbase + s * CHUNK, CHUNK)],
                        acc.at[pl.ds(s * CHUNK, CHUNK)])

    @pl.when(s == NTILES - 1)
    def _():
        pltpu.sync_copy(
            y_hbm.at[pl.ds(zbase + (NTILES - 1) * CHUNK, LAST)],
            acc.at[pl.ds((NTILES - 1) * CHUNK, LAST)])

    pltpu.make_async_copy(idx_hbm.at[c, s, 0], ibufs[0], isems[0]).wait()
    pltpu.async_copy(idx_hbm.at[c, s, 1], ibufs[1], isems[1])
    plsc.subcore_barrier()

    for p in range(NBUF):
        pltpu.async_copy(y_hbm.at[ibufs[0].at[p, 0]], bufs[p], sems[p])

    for k in range(NSB):
        ib = ibufs[k % 2]
        ibn = ibufs[(k + 1) % 2]

        def inner(b2, carry, ib=ib):
            for p in range(NBUF):
                bl = b2 * NBUF + p
                pltpu.make_async_copy(y_hbm.at[ib.at[bl, 0]],
                                      bufs[p], sems[p]).wait()
                pltpu.sync_copy(bufs[p], acc.at[ib.at[bl, 1]], add=True)
                pltpu.async_copy(y_hbm.at[ib.at[bl + NBUF, 0]],
                                 bufs[p], sems[p])
            return carry

        lax.fori_loop(0, (SB - NBUF) // NBUF, inner, 0)
        if k + 1 < NSB:
            pltpu.make_async_copy(idx_hbm.at[c, s, k + 1], ibn,
                                  isems[(k + 1) % 2]).wait()
        for p in range(NBUF):
            bl = SB - NBUF + p
            pltpu.make_async_copy(y_hbm.at[ib.at[bl, 0]],
                                  bufs[p], sems[p]).wait()
            pltpu.sync_copy(bufs[p], acc.at[ib.at[bl, 1]], add=True)
            if k + 1 < NSB:
                pltpu.async_copy(y_hbm.at[ibn.at[p, 0]], bufs[p], sems[p])
        if k + 2 < NSB:
            pltpu.async_copy(idx_hbm.at[c, s, k + 2], ib, isems[k % 2])
    plsc.subcore_barrier()

    @pl.when(s < NTILES - 1)
    def _():
        pltpu.sync_copy(acc.at[pl.ds(s * CHUNK, CHUNK)],
                        out_hbm.at[c, pl.ds(s * CHUNK, CHUNK)])

    @pl.when(s == NTILES - 1)
    def _():
        pltpu.sync_copy(acc.at[pl.ds((NTILES - 1) * CHUNK, LAST)],
                        out_hbm.at[c, pl.ds((NTILES - 1) * CHUNK, LAST)])


@functools.cache
def _make_sc_scatter():
    return pl.kernel(
        _sc_scatter_body,
        out_type=jax.ShapeDtypeStruct((2, N, H), jnp.float32),
        mesh=plsc.VectorSubcoreMesh(core_axis_name="c", subcore_axis_name="s",
                                    num_cores=2, num_subcores=NTILES),
        scratch_types=[
            tuple(pltpu.VMEM((SB, 2, B), jnp.int32) for _ in range(2)),
            tuple(pltpu.SemaphoreType.DMA for _ in range(2)),
            tuple(pltpu.VMEM((B, H), jnp.float32) for _ in range(NBUF)),
            tuple(pltpu.SemaphoreType.DMA for _ in range(NBUF)),
            pltpu.VMEM_SHARED((N, H), jnp.float32),
        ],
    )


def kernel(x, edge_index, edge_type, node_position, W_rel, W_self, b):
    src = edge_index[0]
    dst = edge_index[1]
    tidx = edge_type * N + src
    ge = jnp.stack([tidx, tidx + NROW]).reshape(2, NTILES, NSB, SB, B)
    de = jnp.broadcast_to(dst, (2, E)).reshape(2, NTILES, NSB, SB, B)
    idx = jnp.stack([ge, de], axis=-2)
    w_cat = jnp.concatenate([W_rel, W_self[:, None]], axis=1)

    h = x
    outs = []
    for l in range(L):
        yz = _tc_tables(h, w_cat[l], b[l])
        acc2 = _make_sc_scatter()(yz.reshape(2 * NROW, H), idx)
        h = _tc_relu_concat(acc2)
        outs.append(h)
    node_feature = jnp.concatenate(outs, axis=-1)
    return node_feature, node_position
